# Initial kernel scaffold; baseline (speedup 1.0000x reference)
import jax
import jax.numpy as jnp
from jax.experimental import pallas as pl


def kernel(features, labels):
    raise NotImplementedError("write your pallas kernel here")


# speedup vs baseline: 78.9440x; 78.9440x over previous
import functools

import jax
import jax.numpy as jnp
from jax import lax
from jax.experimental import pallas as pl
from jax.experimental.pallas import tpu as pltpu
from jax.experimental.pallas import tpu_sc as plsc

N = 4096
D = 32
NUM_BINS = 100
NBP = 256
BM = 512
NT = N // BM
NW = 32
CHUNKS = 32
CHUNK = N * N // (NW * CHUNKS)
VECS_PER_CHUNK = CHUNK // 16
UNROLL = 8


def _tc_idx_body(frows_ref, ft_ref, labr_ref, labc_ref, out_ref, mm_ref):
    p = pl.program_id(0)
    i = pl.program_id(1)
    frows = frows_ref[...]
    ft = ft_ref[...]
    sq_r = jnp.sum(frows * frows, axis=1, keepdims=True)
    sq_c = jnp.sum(ft * ft, axis=0, keepdims=True)
    g = jnp.dot(frows, ft, preferred_element_type=jnp.float32)
    c = jnp.maximum(sq_r + sq_c - 2.0 * g, 0.0)

    @pl.when(jnp.logical_and(p == 0, i == 0))
    def _init():
        mm_ref[0] = jnp.inf
        mm_ref[1] = -jnp.inf

    @pl.when(p == 0)
    def _minmax():
        mm_ref[0] = jnp.minimum(mm_ref[0], jnp.min(c))
        mm_ref[1] = jnp.maximum(mm_ref[1], jnp.max(c))

    @pl.when(p == 1)
    def _binify():
        def sqrt_precise(x):
            r = lax.rsqrt(x)
            r = r * (1.5 - 0.5 * x * r * r)
            return x * r

        dist = sqrt_precise(c + 1e-12)
        mind = sqrt_precise(mm_ref[0] + 1e-12)
        maxd = sqrt_precise(mm_ref[1] + 1e-12)
        step = (maxd - mind) * 0.01
        r0 = 1.0 / step
        inv = r0 * (2.0 - step * r0)
        idx = jnp.clip(jnp.floor((dist - mind) * inv).astype(jnp.int32),
                       0, NUM_BINS - 1)
        pos = labr_ref[...] == labc_ref[...]
        lane = lax.broadcasted_iota(jnp.int32, (BM, N), 1) % 16
        out_ref[...] = idx + jnp.where(pos, NUM_BINS, 0) + lane * NBP


def _tc_idx(features, ft, labr, labc):
    return pl.pallas_call(
        _tc_idx_body,
        grid=(2, NT),
        in_specs=[
            pl.BlockSpec((BM, D), lambda p, i: (i, 0)),
            pl.BlockSpec((D, N), lambda p, i: (0, 0)),
            pl.BlockSpec((BM, 1), lambda p, i: (i, 0)),
            pl.BlockSpec((1, N), lambda p, i: (0, 0)),
        ],
        out_specs=pl.BlockSpec((BM, N), lambda p, i: (i * p, 0)),
        out_shape=jax.ShapeDtypeStruct((N, N), jnp.int32),
        scratch_shapes=[pltpu.SMEM((2,), jnp.float32)],
    )(features, ft, labr, labc)


@functools.cache
def _build_sc_hist():
    mesh = plsc.VectorSubcoreMesh(core_axis_name="c", subcore_axis_name="s")
    return functools.partial(
        pl.kernel,
        out_type=jax.ShapeDtypeStruct((NW, 16 * NBP), jnp.float32),
        mesh=mesh,
        scratch_types=[
            pltpu.VMEM((CHUNK,), jnp.int32),
            pltpu.VMEM((16 * NBP,), jnp.float32),
        ],
        compiler_params=pltpu.CompilerParams(needs_layout_passes=False),
    )(_sc_hist_body)


def _sc_hist_body(packed_hbm, out_hbm, chunk_v, hist_v):
    wid = lax.axis_index("s") * 2 + lax.axis_index("c")
    ones = jnp.ones((16,), jnp.float32)
    zeros = jnp.zeros((16,), jnp.float32)

    def zbody(q, _):
        hist_v[pl.ds(q * 16, 16)] = zeros
        return 0
    lax.fori_loop(0, 16 * NBP // 16, zbody, 0)

    def chunk_body(ci, _):
        pltpu.sync_copy(packed_hbm.at[wid, ci], chunk_v)

        def vec_body(vi, _):
            base = vi * (16 * UNROLL)
            for u in range(UNROLL):
                colv = chunk_v[pl.ds(base + u * 16, 16)]
                plsc.addupdate_scatter(hist_v, [colv], ones)
            return 0
        lax.fori_loop(0, VECS_PER_CHUNK // UNROLL, vec_body, 0)
        return 0
    lax.fori_loop(0, CHUNKS, chunk_body, 0)
    pltpu.sync_copy(hist_v, out_hbm.at[wid])


def _tc_loss_body(hist_ref, out_ref):
    h = hist_ref[...]
    tot = jnp.sum(h, axis=0, keepdims=True)
    neg = tot[:, :NUM_BINS]
    pos = tot[:, NUM_BINS:2 * NUM_BINS]
    row = lax.broadcasted_iota(jnp.int32, (NUM_BINS, NUM_BINS), 0)
    col = lax.broadcasted_iota(jnp.int32, (NUM_BINS, NUM_BINS), 1)
    upper = (row <= col).astype(jnp.float32)
    pos_cdf = jnp.dot(pos, upper, preferred_element_type=jnp.float32,
                      precision=lax.Precision.HIGHEST)
    neg_cdf = jnp.dot(neg, upper, preferred_element_type=jnp.float32,
                      precision=lax.Precision.HIGHEST)
    pos_cdf = pos_cdf / jnp.sum(pos)
    neg_cdf = neg_cdf / jnp.sum(neg)
    out_ref[0, 0] = jnp.sum(jnp.abs(pos_cdf - neg_cdf)) / NUM_BINS


def _tc_loss(hist):
    return pl.pallas_call(
        _tc_loss_body,
        out_specs=pl.BlockSpec(memory_space=pltpu.SMEM),
        out_shape=jax.ShapeDtypeStruct((1, 1), jnp.float32),
    )(hist)


def kernel(features, labels):
    ft = features.T
    labr = labels.reshape(N, 1)
    labc = labels.reshape(1, N)
    packed = _tc_idx(features, ft, labr, labc)
    hist = _build_sc_hist()(packed.reshape(NW, CHUNKS, CHUNK))
    loss = _tc_loss(hist.reshape(NW * 16, NBP))
    return loss[0, 0]


# speedup vs baseline: 85.6234x; 1.0846x over previous
import functools

import jax
import jax.numpy as jnp
from jax import lax
from jax.experimental import pallas as pl
from jax.experimental.pallas import tpu as pltpu
from jax.experimental.pallas import tpu_sc as plsc

N = 4096
D = 32
NUM_BINS = 100
NBP = 256
BM = 512
NT = N // BM
NW = 32
CHUNKS = 32
CHUNK = N * N // (NW * CHUNKS)
VECS_PER_CHUNK = CHUNK // 16
UNROLL = 8
BANKS = 64
HIST = BANKS * NBP


def _tc_idx_body(frows_ref, ft_ref, labr_ref, labc_ref, boff_ref, out_ref,
                 mm_ref):
    p = pl.program_id(0)
    i = pl.program_id(1)
    frows = frows_ref[...]
    ft = ft_ref[...]
    sq_r = jnp.sum(frows * frows, axis=1, keepdims=True)
    sq_c = jnp.sum(ft * ft, axis=0, keepdims=True)
    g = jnp.dot(frows, ft, preferred_element_type=jnp.float32)
    c = jnp.maximum(sq_r + sq_c - 2.0 * g, 0.0)

    @pl.when(jnp.logical_and(p == 0, i == 0))
    def _init():
        mm_ref[0] = jnp.inf
        mm_ref[1] = -jnp.inf

    @pl.when(p == 0)
    def _minmax():
        mm_ref[0] = jnp.minimum(mm_ref[0], jnp.min(c))
        mm_ref[1] = jnp.maximum(mm_ref[1], jnp.max(c))

    @pl.when(p == 1)
    def _binify():
        def sqrt_precise(x):
            r = lax.rsqrt(x)
            r = r * (1.5 - 0.5 * x * r * r)
            return x * r

        dist = sqrt_precise(c + 1e-12)
        mind = sqrt_precise(mm_ref[0] + 1e-12)
        maxd = sqrt_precise(mm_ref[1] + 1e-12)
        step = (maxd - mind) * 0.01
        r0 = 1.0 / step
        inv = r0 * (2.0 - step * r0)
        idx = jnp.clip(jnp.floor((dist - mind) * inv).astype(jnp.int32),
                       0, NUM_BINS - 1)
        pos = labr_ref[...] == labc_ref[...]
        out_ref[...] = idx + jnp.where(pos, NUM_BINS, 0) + boff_ref[...]


def _tc_idx(features, ft, labr, labc, boff):
    return pl.pallas_call(
        _tc_idx_body,
        grid=(2, NT),
        in_specs=[
            pl.BlockSpec((BM, D), lambda p, i: (i, 0)),
            pl.BlockSpec((D, N), lambda p, i: (0, 0)),
            pl.BlockSpec((BM, 1), lambda p, i: (i, 0)),
            pl.BlockSpec((1, N), lambda p, i: (0, 0)),
            pl.BlockSpec((1, N), lambda p, i: (0, 0)),
        ],
        out_specs=pl.BlockSpec((BM, N), lambda p, i: (i * p, 0)),
        out_shape=jax.ShapeDtypeStruct((N, N), jnp.int32),
        scratch_shapes=[pltpu.SMEM((2,), jnp.float32)],
    )(features, ft, labr, labc, boff)


@functools.cache
def _build_sc_hist():
    mesh = plsc.VectorSubcoreMesh(core_axis_name="c", subcore_axis_name="s")
    return functools.partial(
        pl.kernel,
        out_type=jax.ShapeDtypeStruct((NW, HIST), jnp.float32),
        mesh=mesh,
        scratch_types=[
            pltpu.VMEM((CHUNK,), jnp.int32),
            pltpu.VMEM((CHUNK,), jnp.int32),
            pltpu.VMEM((HIST,), jnp.float32),
            pltpu.SemaphoreType.DMA,
            pltpu.SemaphoreType.DMA,
        ],
        compiler_params=pltpu.CompilerParams(needs_layout_passes=False),
    )(_sc_hist_body)


def _sc_hist_body(packed_hbm, out_hbm, buf0, buf1, hist_v, sem0, sem1):
    wid = lax.axis_index("s") * 2 + lax.axis_index("c")
    ones = jnp.ones((16,), jnp.float32)
    zeros = jnp.zeros((16,), jnp.float32)

    pltpu.async_copy(packed_hbm.at[wid, 0], buf0, sem0)

    def zbody(q, _):
        hist_v[pl.ds(q * 16, 16)] = zeros
        return 0
    lax.fori_loop(0, HIST // 16, zbody, 0)

    def scatter_chunk(buf):
        def vec_body(vi, _):
            base = vi * (16 * UNROLL)
            for u in range(UNROLL):
                colv = buf[pl.ds(base + u * 16, 16)]
                plsc.addupdate_scatter(hist_v, [colv], ones)
            return 0
        lax.fori_loop(0, VECS_PER_CHUNK // UNROLL, vec_body, 0)

    def pair_body(h, _):
        ci = h * 2
        nxt = pltpu.async_copy(packed_hbm.at[wid, ci + 1], buf1, sem1)
        pltpu.make_async_copy(packed_hbm.at[wid, ci], buf0, sem0).wait()
        scatter_chunk(buf0)

        @pl.when(ci + 2 < CHUNKS)
        def _prefetch():
            pltpu.async_copy(packed_hbm.at[wid, ci + 2], buf0, sem0)
        nxt.wait()
        scatter_chunk(buf1)
        return 0
    lax.fori_loop(0, CHUNKS // 2, pair_body, 0)
    pltpu.sync_copy(hist_v, out_hbm.at[wid])


def _tc_loss_body(hist_ref, out_ref):
    h = hist_ref[...]
    tot = jnp.sum(h, axis=0, keepdims=True)
    neg = tot[:, :NUM_BINS]
    pos = tot[:, NUM_BINS:2 * NUM_BINS]
    row = lax.broadcasted_iota(jnp.int32, (NUM_BINS, NUM_BINS), 0)
    col = lax.broadcasted_iota(jnp.int32, (NUM_BINS, NUM_BINS), 1)
    upper = (row <= col).astype(jnp.float32)
    pos_cdf = jnp.dot(pos, upper, preferred_element_type=jnp.float32,
                      precision=lax.Precision.HIGHEST)
    neg_cdf = jnp.dot(neg, upper, preferred_element_type=jnp.float32,
                      precision=lax.Precision.HIGHEST)
    pos_cdf = pos_cdf / jnp.sum(pos)
    neg_cdf = neg_cdf / jnp.sum(neg)
    out_ref[0, 0] = jnp.sum(jnp.abs(pos_cdf - neg_cdf)) / NUM_BINS


def _tc_loss(hist):
    return pl.pallas_call(
        _tc_loss_body,
        out_specs=pl.BlockSpec(memory_space=pltpu.SMEM),
        out_shape=jax.ShapeDtypeStruct((1, 1), jnp.float32),
    )(hist)


def kernel(features, labels):
    ft = features.T
    labr = labels.reshape(N, 1)
    labc = labels.reshape(1, N)
    col = jnp.arange(N, dtype=jnp.int32)
    boff = (((col % 16) + 16 * ((col // 16) % 4)) * NBP).reshape(1, N)
    packed = _tc_idx(features, ft, labr, labc, boff)
    hist = _build_sc_hist()(packed.reshape(NW, CHUNKS, CHUNK))
    loss = _tc_loss(hist.reshape(NW * BANKS, NBP))
    return loss[0, 0]


# speedup vs baseline: 123.3194x; 1.4403x over previous
import functools

import jax
import jax.numpy as jnp
from jax import lax
from jax.experimental import pallas as pl
from jax.experimental.pallas import tpu as pltpu
from jax.experimental.pallas import tpu_sc as plsc

N = 4096
D = 32
NUM_BINS = 100
NBP = 256
BM = 512
NT = N // BM
NTILES = NT * (NT + 1) // 2
DIAG_TILES = tuple(bi * NT + bi - (bi * (bi + 1)) // 2 for bi in range(NT))
NW = 32
TOTAL = NTILES * BM * BM
CHUNK = 16384
CHUNKS = TOTAL // (NW * CHUNK)
CHUNKS_PER_TILE = BM * BM // CHUNK
VECS_PER_CHUNK = CHUNK // 16
UNROLL = 8
BANKS = 64
HIST = BANKS * NBP


def _tc_idx_body(frows_ref, fcols_ref, labr_ref, labc_ref, boff_ref, out_ref,
                 mm_ref):
    p = pl.program_id(0)
    bi = pl.program_id(1)
    bj = pl.program_id(2)
    active = bj >= bi

    @pl.when(jnp.logical_and(p == 0, jnp.logical_and(bi == 0, bj == 0)))
    def _init():
        mm_ref[0] = jnp.inf
        mm_ref[1] = -jnp.inf

    @pl.when(jnp.logical_and(p == 0, active))
    def _minmax():
        frows = frows_ref[...]
        fcols = fcols_ref[...]
        sq_r = jnp.sum(frows * frows, axis=1, keepdims=True)
        sq_c = jnp.sum(fcols * fcols, axis=0, keepdims=True)
        g = jnp.dot(frows, fcols, preferred_element_type=jnp.float32)
        c = jnp.maximum(sq_r + sq_c - 2.0 * g, 0.0)
        mm_ref[0] = jnp.minimum(mm_ref[0], jnp.min(c))
        mm_ref[1] = jnp.maximum(mm_ref[1], jnp.max(c))

    @pl.when(jnp.logical_and(p == 1, active))
    def _binify():
        frows = frows_ref[...]
        fcols = fcols_ref[...]
        sq_r = jnp.sum(frows * frows, axis=1, keepdims=True)
        sq_c = jnp.sum(fcols * fcols, axis=0, keepdims=True)
        g = jnp.dot(frows, fcols, preferred_element_type=jnp.float32)
        c = jnp.maximum(sq_r + sq_c - 2.0 * g, 0.0)

        def sqrt_precise(x):
            r = lax.rsqrt(x)
            r = r * (1.5 - 0.5 * x * r * r)
            return x * r

        dist = sqrt_precise(c + 1e-12)
        mind = sqrt_precise(mm_ref[0] + 1e-12)
        maxd = sqrt_precise(mm_ref[1] + 1e-12)
        step = (maxd - mind) * 0.01
        r0 = 1.0 / step
        inv = r0 * (2.0 - step * r0)
        idx = jnp.clip(jnp.floor((dist - mind) * inv).astype(jnp.int32),
                       0, NUM_BINS - 1)
        pos = labr_ref[...] == labc_ref[...]
        out_ref[...] = (idx + jnp.where(pos, NUM_BINS, 0)
                        + boff_ref[...])[None]


def _tile_index(bi, bj):
    up = bi * NT + bj - (bi * (bi + 1)) // 2
    dg = bi * NT + bi - (bi * (bi + 1)) // 2
    return jnp.where(bj >= bi, up, dg)


def _tc_idx(features, ft, labr, labc, boff):
    return pl.pallas_call(
        _tc_idx_body,
        grid=(2, NT, NT),
        in_specs=[
            pl.BlockSpec((BM, D), lambda p, bi, bj: (bi, 0)),
            pl.BlockSpec((D, BM), lambda p, bi, bj: (0, bj)),
            pl.BlockSpec((BM, 1), lambda p, bi, bj: (bi, 0)),
            pl.BlockSpec((1, BM), lambda p, bi, bj: (0, bj)),
            pl.BlockSpec((1, BM), lambda p, bi, bj: (0, 0)),
        ],
        out_specs=pl.BlockSpec(
            (1, BM, BM), lambda p, bi, bj: (_tile_index(bi, bj), 0, 0)),
        out_shape=jax.ShapeDtypeStruct((NTILES, BM, BM), jnp.int32),
        scratch_shapes=[pltpu.SMEM((2,), jnp.float32)],
    )(features, ft, labr, labc, boff)


@functools.cache
def _build_sc_hist():
    mesh = plsc.VectorSubcoreMesh(core_axis_name="c", subcore_axis_name="s")
    return functools.partial(
        pl.kernel,
        out_type=jax.ShapeDtypeStruct((NW, HIST), jnp.float32),
        mesh=mesh,
        scratch_types=[
            pltpu.VMEM((CHUNK,), jnp.int32),
            pltpu.VMEM((CHUNK,), jnp.int32),
            pltpu.VMEM((HIST,), jnp.float32),
            pltpu.SemaphoreType.DMA,
            pltpu.SemaphoreType.DMA,
        ],
        compiler_params=pltpu.CompilerParams(needs_layout_passes=False),
    )(_sc_hist_body)


def _sc_hist_body(packed_hbm, out_hbm, buf0, buf1, hist_v, sem0, sem1):
    wid = lax.axis_index("s") * 2 + lax.axis_index("c")
    ones = jnp.ones((16,), jnp.float32)
    zeros = jnp.zeros((16,), jnp.float32)

    pltpu.async_copy(packed_hbm.at[wid, 0], buf0, sem0)

    def zbody(q, _):
        hist_v[pl.ds(q * 16, 16)] = zeros
        return 0
    lax.fori_loop(0, HIST // 16, zbody, 0)

    def chunk_weight(ci):
        tile = (wid * CHUNKS + ci) // CHUNKS_PER_TILE
        isdiag = tile == DIAG_TILES[0]
        for t in DIAG_TILES[1:]:
            isdiag = jnp.logical_or(isdiag, tile == t)
        return jnp.where(isdiag, 1.0, 2.0)

    def scatter_chunk(buf, ci):
        wvec = ones * chunk_weight(ci)

        def vec_body(vi, _):
            base = vi * (16 * UNROLL)
            for u in range(UNROLL):
                colv = buf[pl.ds(base + u * 16, 16)]
                plsc.addupdate_scatter(hist_v, [colv], wvec)
            return 0
        lax.fori_loop(0, VECS_PER_CHUNK // UNROLL, vec_body, 0)

    def pair_body(h, _):
        ci = h * 2
        nxt = pltpu.async_copy(packed_hbm.at[wid, ci + 1], buf1, sem1)
        pltpu.make_async_copy(packed_hbm.at[wid, ci], buf0, sem0).wait()
        scatter_chunk(buf0, ci)

        @pl.when(ci + 2 < CHUNKS)
        def _prefetch():
            pltpu.async_copy(packed_hbm.at[wid, ci + 2], buf0, sem0)
        nxt.wait()
        scatter_chunk(buf1, ci + 1)
        return 0
    lax.fori_loop(0, CHUNKS // 2, pair_body, 0)
    pltpu.sync_copy(hist_v, out_hbm.at[wid])


def _tc_loss_body(hist_ref, out_ref):
    h = hist_ref[...]
    tot = jnp.sum(h, axis=0, keepdims=True)
    neg = tot[:, :NUM_BINS]
    pos = tot[:, NUM_BINS:2 * NUM_BINS]
    row = lax.broadcasted_iota(jnp.int32, (NUM_BINS, NUM_BINS), 0)
    col = lax.broadcasted_iota(jnp.int32, (NUM_BINS, NUM_BINS), 1)
    upper = (row <= col).astype(jnp.float32)
    pos_cdf = jnp.dot(pos, upper, preferred_element_type=jnp.float32,
                      precision=lax.Precision.HIGHEST)
    neg_cdf = jnp.dot(neg, upper, preferred_element_type=jnp.float32,
                      precision=lax.Precision.HIGHEST)
    pos_cdf = pos_cdf / jnp.sum(pos)
    neg_cdf = neg_cdf / jnp.sum(neg)
    out_ref[0, 0] = jnp.sum(jnp.abs(pos_cdf - neg_cdf)) / NUM_BINS


def _tc_loss(hist):
    return pl.pallas_call(
        _tc_loss_body,
        out_specs=pl.BlockSpec(memory_space=pltpu.SMEM),
        out_shape=jax.ShapeDtypeStruct((1, 1), jnp.float32),
    )(hist)


def kernel(features, labels):
    ft = features.T
    labr = labels.reshape(N, 1)
    labc = labels.reshape(1, N)
    col = jnp.arange(BM, dtype=jnp.int32)
    boff = (((col % 16) + 16 * ((col // 16) % 4)) * NBP).reshape(1, BM)
    packed = _tc_idx(features, ft, labr, labc, boff)
    hist = _build_sc_hist()(packed.reshape(NW, CHUNKS, CHUNK))
    loss = _tc_loss(hist.reshape(NW * BANKS, NBP))
    return loss[0, 0]


# speedup vs baseline: 165.1935x; 1.3396x over previous
import functools

import jax
import jax.numpy as jnp
from jax import lax
from jax.experimental import pallas as pl
from jax.experimental.pallas import tpu as pltpu
from jax.experimental.pallas import tpu_sc as plsc

N = 4096
D = 32
NUM_BINS = 100
NBP = 256
BM = 512
NT = N // BM
NTILES = NT * (NT + 1) // 2
DIAG_TILES = tuple(bi * NT + bi - (bi * (bi + 1)) // 2 for bi in range(NT))
NW = 32
TOTAL = NTILES * BM * BM
CHUNK = 16384
CHUNKS = TOTAL // (NW * CHUNK)
CHUNKS_PER_TILE = BM * BM // CHUNK
VECS_PER_CHUNK = CHUNK // 16
UNROLL = 8
BANKS = 64
HIST = BANKS * NBP


def _tc_idx_body(frows_ref, fcols_ref, labr_ref, labc_ref, boff_ref, out_ref,
                 mm_ref):
    p = pl.program_id(0)
    bi = pl.program_id(1)
    bj = pl.program_id(2)
    active = bj >= bi

    @pl.when(jnp.logical_and(p == 0, jnp.logical_and(bi == 0, bj == 0)))
    def _init():
        mm_ref[0] = jnp.inf
        mm_ref[1] = -jnp.inf

    @pl.when(jnp.logical_and(p == 0, active))
    def _minmax():
        frows = frows_ref[...]
        fcols = fcols_ref[...]
        sq_r = jnp.sum(frows * frows, axis=1, keepdims=True)
        sq_c = jnp.sum(fcols * fcols, axis=0, keepdims=True)
        g = jnp.dot(frows, fcols, preferred_element_type=jnp.float32)
        c = jnp.maximum(sq_r + sq_c - 2.0 * g, 0.0)
        mm_ref[0] = jnp.minimum(mm_ref[0], jnp.min(c))
        mm_ref[1] = jnp.maximum(mm_ref[1], jnp.max(c))

    @pl.when(jnp.logical_and(p == 1, active))
    def _binify():
        frows = frows_ref[...]
        fcols = fcols_ref[...]
        sq_r = jnp.sum(frows * frows, axis=1, keepdims=True)
        sq_c = jnp.sum(fcols * fcols, axis=0, keepdims=True)
        g = jnp.dot(frows, fcols, preferred_element_type=jnp.float32)
        c = jnp.maximum(sq_r + sq_c - 2.0 * g, 0.0)

        def sqrt_precise(x):
            r = lax.rsqrt(x)
            r = r * (1.5 - 0.5 * x * r * r)
            return x * r

        dist = sqrt_precise(c + 1e-12)
        mind = sqrt_precise(mm_ref[0] + 1e-12)
        maxd = sqrt_precise(mm_ref[1] + 1e-12)
        step = (maxd - mind) * 0.01
        r0 = 1.0 / step
        inv = r0 * (2.0 - step * r0)
        idx = jnp.clip(jnp.floor((dist - mind) * inv).astype(jnp.int32),
                       0, NUM_BINS - 1)
        pos = labr_ref[...] == labc_ref[...]
        out_ref[...] = (idx + jnp.where(pos, NUM_BINS, 0)
                        + boff_ref[...])[None]


def _tile_index(bi, bj):
    up = bi * NT + bj - (bi * (bi + 1)) // 2
    dg = bi * NT + bi - (bi * (bi + 1)) // 2
    return jnp.where(bj >= bi, up, dg)


def _tc_idx(features, ft, labr, labc, boff):
    return pl.pallas_call(
        _tc_idx_body,
        grid=(2, NT, NT),
        in_specs=[
            pl.BlockSpec((BM, D), lambda p, bi, bj: (bi, 0)),
            pl.BlockSpec((D, BM), lambda p, bi, bj: (0, bj)),
            pl.BlockSpec((BM, 1), lambda p, bi, bj: (bi, 0)),
            pl.BlockSpec((1, BM), lambda p, bi, bj: (0, bj)),
            pl.BlockSpec((1, BM), lambda p, bi, bj: (0, 0)),
        ],
        out_specs=pl.BlockSpec(
            (1, BM, BM), lambda p, bi, bj: (_tile_index(bi, bj), 0, 0)),
        out_shape=jax.ShapeDtypeStruct((NTILES, BM, BM), jnp.int32),
        scratch_shapes=[pltpu.SMEM((2,), jnp.float32)],
    )(features, ft, labr, labc, boff)


@functools.cache
def _build_sc_hist():
    mesh = plsc.VectorSubcoreMesh(core_axis_name="c", subcore_axis_name="s")
    return functools.partial(
        pl.kernel,
        out_type=jax.ShapeDtypeStruct((NW, HIST), jnp.float32),
        mesh=mesh,
        scratch_types=[
            pltpu.VMEM((CHUNK,), jnp.int32),
            pltpu.VMEM((CHUNK,), jnp.int32),
            pltpu.VMEM((HIST,), jnp.float32),
            pltpu.SemaphoreType.DMA,
            pltpu.SemaphoreType.DMA,
        ],
        compiler_params=pltpu.CompilerParams(needs_layout_passes=False),
    )(_sc_hist_body)


def _sc_hist_body(packed_hbm, out_hbm, buf0, buf1, hist_v, sem0, sem1):
    wid = lax.axis_index("s") * 2 + lax.axis_index("c")
    ones = jnp.ones((16,), jnp.float32)
    zeros = jnp.zeros((16,), jnp.float32)

    pltpu.async_copy(packed_hbm.at[wid, 0], buf0, sem0)

    def zbody(q, _):
        hist_v[pl.ds(q * 16, 16)] = zeros
        return 0
    lax.fori_loop(0, HIST // 16, zbody, 0)

    def chunk_weight(ci):
        tile = (wid * CHUNKS + ci) // CHUNKS_PER_TILE
        isdiag = tile == DIAG_TILES[0]
        for t in DIAG_TILES[1:]:
            isdiag = jnp.logical_or(isdiag, tile == t)
        return jnp.where(isdiag, 1.0, 2.0)

    def scatter_chunk(buf, ci):
        wvec = ones * chunk_weight(ci)

        @plsc.parallel_loop(0, CHUNK, step=16, unroll=UNROLL)
        def _vec_body(i):
            colv = buf[pl.ds(i, 16)]
            plsc.addupdate_scatter(hist_v, [colv], wvec)

    def pair_body(h, _):
        ci = h * 2
        nxt = pltpu.async_copy(packed_hbm.at[wid, ci + 1], buf1, sem1)
        pltpu.make_async_copy(packed_hbm.at[wid, ci], buf0, sem0).wait()
        scatter_chunk(buf0, ci)

        @pl.when(ci + 2 < CHUNKS)
        def _prefetch():
            pltpu.async_copy(packed_hbm.at[wid, ci + 2], buf0, sem0)
        nxt.wait()
        scatter_chunk(buf1, ci + 1)
        return 0
    lax.fori_loop(0, CHUNKS // 2, pair_body, 0)
    pltpu.sync_copy(hist_v, out_hbm.at[wid])


def _tc_loss_body(hist_ref, out_ref):
    h = hist_ref[...]
    tot = jnp.sum(h, axis=0, keepdims=True)
    neg = tot[:, :NUM_BINS]
    pos = tot[:, NUM_BINS:2 * NUM_BINS]
    row = lax.broadcasted_iota(jnp.int32, (NUM_BINS, NUM_BINS), 0)
    col = lax.broadcasted_iota(jnp.int32, (NUM_BINS, NUM_BINS), 1)
    upper = (row <= col).astype(jnp.float32)
    pos_cdf = jnp.dot(pos, upper, preferred_element_type=jnp.float32,
                      precision=lax.Precision.HIGHEST)
    neg_cdf = jnp.dot(neg, upper, preferred_element_type=jnp.float32,
                      precision=lax.Precision.HIGHEST)
    pos_cdf = pos_cdf / jnp.sum(pos)
    neg_cdf = neg_cdf / jnp.sum(neg)
    out_ref[0, 0] = jnp.sum(jnp.abs(pos_cdf - neg_cdf)) / NUM_BINS


def _tc_loss(hist):
    return pl.pallas_call(
        _tc_loss_body,
        out_specs=pl.BlockSpec(memory_space=pltpu.SMEM),
        out_shape=jax.ShapeDtypeStruct((1, 1), jnp.float32),
    )(hist)


def kernel(features, labels):
    ft = features.T
    labr = labels.reshape(N, 1)
    labc = labels.reshape(1, N)
    col = jnp.arange(BM, dtype=jnp.int32)
    boff = (((col % 16) + 16 * ((col // 16) % 4)) * NBP).reshape(1, BM)
    packed = _tc_idx(features, ft, labr, labc, boff)
    hist = _build_sc_hist()(packed.reshape(NW, CHUNKS, CHUNK))
    loss = _tc_loss(hist.reshape(NW * BANKS, NBP))
    return loss[0, 0]


# speedup vs baseline: 190.2863x; 1.1519x over previous
import functools

import jax
import jax.numpy as jnp
from jax import lax
from jax.experimental import pallas as pl
from jax.experimental.pallas import tpu as pltpu
from jax.experimental.pallas import tpu_sc as plsc

N = 4096
D = 32
NUM_BINS = 100
NBP = 256
BM = 1024
NT = N // BM
NTILES = NT * (NT + 1) // 2
DIAG_TILES = tuple(bi * NT + bi - (bi * (bi + 1)) // 2 for bi in range(NT))
NW = 32
TOTAL = NTILES * BM * BM
CHUNK = 16384
CHUNKS = TOTAL // (NW * CHUNK)
CHUNKS_PER_TILE = BM * BM // CHUNK
VECS_PER_CHUNK = CHUNK // 16
UNROLL = 8
BANKS = 64
HIST = BANKS * NBP


def _tc_idx_body(frows_ref, fcols_ref, labr_ref, labc_ref, boff_ref, out_ref,
                 mm_ref):
    p = pl.program_id(0)
    bi = pl.program_id(1)
    bj = pl.program_id(2)
    active = bj >= bi

    @pl.when(jnp.logical_and(p == 0, jnp.logical_and(bi == 0, bj == 0)))
    def _init():
        mm_ref[0] = jnp.inf
        mm_ref[1] = -jnp.inf

    @pl.when(jnp.logical_and(p == 0, active))
    def _minmax():
        frows = frows_ref[...]
        fcols = fcols_ref[...]
        sq_r = jnp.sum(frows * frows, axis=1, keepdims=True)
        sq_c = jnp.sum(fcols * fcols, axis=0, keepdims=True)
        g = jnp.dot(frows, fcols, preferred_element_type=jnp.float32)
        c = jnp.maximum(sq_r + sq_c - 2.0 * g, 0.0)
        mm_ref[0] = jnp.minimum(mm_ref[0], jnp.min(c))
        mm_ref[1] = jnp.maximum(mm_ref[1], jnp.max(c))

    @pl.when(jnp.logical_and(p == 1, active))
    def _binify():
        frows = frows_ref[...]
        fcols = fcols_ref[...]
        sq_r = jnp.sum(frows * frows, axis=1, keepdims=True)
        sq_c = jnp.sum(fcols * fcols, axis=0, keepdims=True)
        g = jnp.dot(frows, fcols, preferred_element_type=jnp.float32)
        c = jnp.maximum(sq_r + sq_c - 2.0 * g, 0.0)

        def sqrt_precise(x):
            r = lax.rsqrt(x)
            r = r * (1.5 - 0.5 * x * r * r)
            return x * r

        dist = sqrt_precise(c + 1e-12)
        mind = sqrt_precise(mm_ref[0] + 1e-12)
        maxd = sqrt_precise(mm_ref[1] + 1e-12)
        step = (maxd - mind) * 0.01
        r0 = 1.0 / step
        inv = r0 * (2.0 - step * r0)
        idx = jnp.clip(jnp.floor((dist - mind) * inv).astype(jnp.int32),
                       0, NUM_BINS - 1)
        pos = labr_ref[...] == labc_ref[...]
        out_ref[...] = (idx + jnp.where(pos, NUM_BINS, 0)
                        + boff_ref[...])[None]


def _tile_index(bi, bj):
    up = bi * NT + bj - (bi * (bi + 1)) // 2
    dg = bi * NT + bi - (bi * (bi + 1)) // 2
    return jnp.where(bj >= bi, up, dg)


def _tc_idx(features, ft, labr, labc, boff):
    return pl.pallas_call(
        _tc_idx_body,
        grid=(2, NT, NT),
        in_specs=[
            pl.BlockSpec((BM, D), lambda p, bi, bj: (bi, 0)),
            pl.BlockSpec((D, BM), lambda p, bi, bj: (0, bj)),
            pl.BlockSpec((BM, 1), lambda p, bi, bj: (bi, 0)),
            pl.BlockSpec((1, BM), lambda p, bi, bj: (0, bj)),
            pl.BlockSpec((1, BM), lambda p, bi, bj: (0, 0)),
        ],
        out_specs=pl.BlockSpec(
            (1, BM, BM), lambda p, bi, bj: (_tile_index(bi, bj), 0, 0)),
        out_shape=jax.ShapeDtypeStruct((NTILES, BM, BM), jnp.int32),
        scratch_shapes=[pltpu.SMEM((2,), jnp.float32)],
    )(features, ft, labr, labc, boff)


@functools.cache
def _build_sc_hist():
    mesh = plsc.VectorSubcoreMesh(core_axis_name="c", subcore_axis_name="s")
    return functools.partial(
        pl.kernel,
        out_type=jax.ShapeDtypeStruct((NW, HIST), jnp.float32),
        mesh=mesh,
        scratch_types=[
            pltpu.VMEM((CHUNK,), jnp.int32),
            pltpu.VMEM((CHUNK,), jnp.int32),
            pltpu.VMEM((HIST,), jnp.float32),
            pltpu.SemaphoreType.DMA,
            pltpu.SemaphoreType.DMA,
        ],
        compiler_params=pltpu.CompilerParams(needs_layout_passes=False),
    )(_sc_hist_body)


def _sc_hist_body(packed_hbm, out_hbm, buf0, buf1, hist_v, sem0, sem1):
    wid = lax.axis_index("s") * 2 + lax.axis_index("c")
    ones = jnp.ones((16,), jnp.float32)
    zeros = jnp.zeros((16,), jnp.float32)

    pltpu.async_copy(packed_hbm.at[wid, 0], buf0, sem0)

    def zbody(q, _):
        hist_v[pl.ds(q * 16, 16)] = zeros
        return 0
    lax.fori_loop(0, HIST // 16, zbody, 0)

    def chunk_weight(ci):
        tile = (wid * CHUNKS + ci) // CHUNKS_PER_TILE
        isdiag = tile == DIAG_TILES[0]
        for t in DIAG_TILES[1:]:
            isdiag = jnp.logical_or(isdiag, tile == t)
        return jnp.where(isdiag, 1.0, 2.0)

    def scatter_chunk(buf, ci):
        wvec = ones * chunk_weight(ci)

        @plsc.parallel_loop(0, CHUNK, step=16, unroll=UNROLL)
        def _vec_body(i):
            colv = buf[pl.ds(i, 16)]
            plsc.addupdate_scatter(hist_v, [colv], wvec)

    def pair_body(h, _):
        ci = h * 2
        nxt = pltpu.async_copy(packed_hbm.at[wid, ci + 1], buf1, sem1)
        pltpu.make_async_copy(packed_hbm.at[wid, ci], buf0, sem0).wait()
        scatter_chunk(buf0, ci)

        @pl.when(ci + 2 < CHUNKS)
        def _prefetch():
            pltpu.async_copy(packed_hbm.at[wid, ci + 2], buf0, sem0)
        nxt.wait()
        scatter_chunk(buf1, ci + 1)
        return 0
    lax.fori_loop(0, CHUNKS // 2, pair_body, 0)
    pltpu.sync_copy(hist_v, out_hbm.at[wid])


def _tc_loss_body(hist_ref, out_ref):
    h = hist_ref[...]
    tot = jnp.sum(h, axis=0, keepdims=True)
    neg = tot[:, :NUM_BINS]
    pos = tot[:, NUM_BINS:2 * NUM_BINS]
    row = lax.broadcasted_iota(jnp.int32, (NUM_BINS, NUM_BINS), 0)
    col = lax.broadcasted_iota(jnp.int32, (NUM_BINS, NUM_BINS), 1)
    upper = (row <= col).astype(jnp.float32)
    pos_cdf = jnp.dot(pos, upper, preferred_element_type=jnp.float32,
                      precision=lax.Precision.HIGHEST)
    neg_cdf = jnp.dot(neg, upper, preferred_element_type=jnp.float32,
                      precision=lax.Precision.HIGHEST)
    pos_cdf = pos_cdf / jnp.sum(pos)
    neg_cdf = neg_cdf / jnp.sum(neg)
    out_ref[0, 0] = jnp.sum(jnp.abs(pos_cdf - neg_cdf)) / NUM_BINS


def _tc_loss(hist):
    return pl.pallas_call(
        _tc_loss_body,
        out_specs=pl.BlockSpec(memory_space=pltpu.SMEM),
        out_shape=jax.ShapeDtypeStruct((1, 1), jnp.float32),
    )(hist)


def kernel(features, labels):
    ft = features.T
    labr = labels.reshape(N, 1)
    labc = labels.reshape(1, N)
    col = jnp.arange(BM, dtype=jnp.int32)
    boff = (((col % 16) + 16 * ((col // 16) % 4)) * NBP).reshape(1, BM)
    packed = _tc_idx(features, ft, labr, labc, boff)
    hist = _build_sc_hist()(packed.reshape(NW, CHUNKS, CHUNK))
    loss = _tc_loss(hist.reshape(NW * BANKS, NBP))
    return loss[0, 0]
